# baseline (speedup 1.0000x reference)
import functools

import jax
import jax.numpy as jnp
from jax import lax
from jax.experimental import pallas as pl
from jax.experimental.pallas import tpu as pltpu
from jax.experimental.pallas import tpu_sc as plsc

NUM_NODES = 1000000
D = 64
MAX_NORM = 1.0
L = 16
NC = 2
NS = 16
NW = NC * NS
B = 16384 * 50
IDX_BLK = 128
CHUNK = 512
SUB = CHUNK // IDX_BLK
ROWS_PER_W = B // NW
CHUNKS_PER_W = ROWS_PER_W // CHUNK


def _rsqrt(s):
    bits = plsc.bitcast(s, jnp.int32)
    r = plsc.bitcast(jnp.int32(0x5F3759DF) - (bits >> 1), jnp.float32)
    half = s * 0.5
    for _ in range(3):
        r = r * (1.5 - half * r * r)
    return r


def _sc_body(idx_hbm, table_hbm, out_hbm, idx_v, rows_v, sem):
    wid = lax.axis_index("s") * NC + lax.axis_index("c")
    lanes = lax.iota(jnp.int32, L)

    def chunk_body(i, carry):
        blk0 = wid * (CHUNKS_PER_W * SUB) + i * SUB
        row0 = blk0 * IDX_BLK
        pltpu.sync_copy(idx_hbm.at[pl.ds(blk0, SUB)], idx_v)
        copies = [
            pltpu.async_copy(
                table_hbm.at[idx_v.at[g]],
                rows_v.at[pl.ds(g * IDX_BLK, IDX_BLK)],
                sem,
            )
            for g in range(SUB)
        ]
        for c in copies:
            c.wait()

        def grp_body(g, carry2):
            row_idx = g * L + lanes
            acc = jnp.zeros((L,), jnp.float32)
            for k in range(D):
                col = jnp.full((L,), k, jnp.int32)
                v = plsc.load_gather(rows_v, [row_idx, col])
                acc = acc + v * v
            r = _rsqrt(acc)
            norm = acc * r
            scale = jnp.where(
                acc > MAX_NORM * MAX_NORM, MAX_NORM / (norm + 1e-7), 1.0
            )
            for k in range(D):
                col = jnp.full((L,), k, jnp.int32)
                v = plsc.load_gather(rows_v, [row_idx, col])
                plsc.store_scatter(rows_v, [row_idx, col], v * scale)
            return carry2

        lax.fori_loop(0, CHUNK // L, grp_body, 0)
        pltpu.sync_copy(rows_v, out_hbm.at[pl.ds(row0, CHUNK)])
        return carry

    lax.fori_loop(0, CHUNKS_PER_W, chunk_body, 0)


@jax.jit
def kernel(node_idx, table):
    idx_flat = node_idx.reshape(B // IDX_BLK, IDX_BLK).astype(jnp.int32)
    mesh = plsc.VectorSubcoreMesh(core_axis_name="c", subcore_axis_name="s")
    out = pl.kernel(
        _sc_body,
        out_type=jax.ShapeDtypeStruct((B, D), jnp.float32),
        mesh=mesh,
        compiler_params=pltpu.CompilerParams(
            needs_layout_passes=False, use_tc_tiling_on_sc=False
        ),
        scratch_types=[
            pltpu.VMEM((SUB, IDX_BLK), jnp.int32),
            pltpu.VMEM((CHUNK, D), jnp.float32),
            pltpu.SemaphoreType.DMA,
        ],
    )(idx_flat, table)
    return out.reshape(node_idx.shape[0], node_idx.shape[1], D)


# speedup vs baseline: 2.5557x; 2.5557x over previous
import functools

import jax
import jax.numpy as jnp
from jax import lax
from jax.experimental import pallas as pl
from jax.experimental.pallas import tpu as pltpu
from jax.experimental.pallas import tpu_sc as plsc

D = 64
MAX_NORM = 1.0
L = 16
NC = 2
NS = 16
NW = NC * NS
N_ROWS = 16384
K = 50
NB_TOT = N_ROWS // 128
NB_PER_W = NB_TOT // NW
TILES = K * NB_PER_W
NBUF = 4
GROUPS = 128 // L
SPAD = 17
TPAD = 129


def _rsqrt(s):
    bits = plsc.bitcast(s, jnp.int32)
    r = plsc.bitcast(jnp.int32(0x5F3759DF) - (bits >> 1), jnp.float32)
    half = s * 0.5
    for _ in range(3):
        r = r * (1.5 - half * r * r)
    return r


def _sc_body(idx_hbm, table_hbm, out_hbm, ibufs, bufs, tbufs, s_flat,
             gsems, osems):
    wid = lax.axis_index("s") * NC + lax.axis_index("c")
    nb_base = wid * NB_PER_W
    lanes = lax.iota(jnp.int32, L)
    lanes_pad = lanes * SPAD
    lanes_t = lanes * TPAD

    def tile_kn(t):
        k = t % K
        nb = t // K
        return k, nb

    def gather_descs(t, b):
        return [
            pltpu.make_async_copy(
                table_hbm.at[ibufs[b]],
                bufs[b],
                gsems[b],
            )
        ]

    def launch_gather(t, b):
        k, nb = tile_kn(t)
        pltpu.sync_copy(
            idx_hbm.at[k, pl.ds((nb_base + nb) * 128, 128)], ibufs[b]
        )
        for dsc in gather_descs(t, b):
            dsc.start()

    def wait_gather(t, b):
        for dsc in gather_descs(t, b):
            dsc.wait()

    def out_descs(t, b):
        k, nb = tile_kn(t)
        return [
            pltpu.make_async_copy(
                tbufs[b].at[pl.ds(db * 8, 8), pl.ds(0, 128)],
                out_hbm.at[k * 8 + db, nb_base + nb],
                osems[b],
            )
            for db in range(8)
        ]

    def compute(b):
        rows = bufs[b]
        tbuf = tbufs[b]

        def grp(g, carry):
            q0 = g * L
            for j in range(L):
                q = q0 + j
                v0 = rows[q, pl.ds(0, 16)]
                v1 = rows[q, pl.ds(16, 16)]
                v2 = rows[q, pl.ds(32, 16)]
                v3 = rows[q, pl.ds(48, 16)]
                s = v0 * v0 + v1 * v1 + v2 * v2 + v3 * v3
                plsc.store_scatter(s_flat, [lanes + (j * SPAD)], s)
            acc = plsc.load_gather(s_flat, [lanes_pad])
            for cc in range(1, L):
                acc = acc + plsc.load_gather(s_flat, [lanes_pad + cc])
            r = _rsqrt(acc)
            norm = acc * r
            scale = jnp.where(
                acc > MAX_NORM * MAX_NORM, MAX_NORM / (norm + 1e-7), 1.0
            )
            for j in range(L):
                q = q0 + j
                sj = scale[j]
                for c4 in range(4):
                    v = rows[q, pl.ds(c4 * 16, 16)]
                    plsc.store_scatter(
                        tbuf,
                        [lanes + (c4 * 16), jnp.full((L,), q, jnp.int32)],
                        v * sj,
                    )
            return carry

        lax.fori_loop(0, GROUPS, grp, 0)

    launch_gather(0, 0)
    launch_gather(1, 1)

    def quad(p, carry):
        for b in range(NBUF):
            t = p * NBUF + b
            wait_gather(t, b)
            compute(b)
            for dsc in out_descs(t, b):
                dsc.start()
            nb2 = (b + 2) % NBUF

            @pl.when(t >= 2)
            def _():
                for dsc in out_descs(t - 2, nb2):
                    dsc.wait()

            @pl.when(t + 2 < TILES)
            def _():
                launch_gather(t + 2, nb2)

        return carry

    lax.fori_loop(0, TILES // NBUF, quad, 0)
    for t in (TILES - 2, TILES - 1):
        for dsc in out_descs(t, t % NBUF):
            dsc.wait()


@jax.jit
def kernel(node_idx, table):
    mesh = plsc.VectorSubcoreMesh(core_axis_name="c", subcore_axis_name="s")
    out4 = pl.kernel(
        _sc_body,
        out_type=jax.ShapeDtypeStruct((K * 8, NB_TOT, 8, 128), jnp.float32),
        mesh=mesh,
        compiler_params=pltpu.CompilerParams(
            needs_layout_passes=False, use_tc_tiling_on_sc=False
        ),
        scratch_types=[
            [pltpu.VMEM((128,), jnp.int32) for _ in range(NBUF)],
            [pltpu.VMEM((128, D), jnp.float32) for _ in range(NBUF)],
            [pltpu.VMEM((D, TPAD), jnp.float32) for _ in range(NBUF)],
            pltpu.VMEM((L * SPAD,), jnp.float32),
            [pltpu.SemaphoreType.DMA for _ in range(NBUF)],
            [pltpu.SemaphoreType.DMA for _ in range(NBUF)],
        ],
    )(jnp.transpose(node_idx).astype(jnp.int32), table)
    out5 = out4.reshape(K, 8, NB_TOT, 8, 128)
    return out5.transpose(2, 4, 0, 1, 3).reshape(N_ROWS, K, D)


# speedup vs baseline: 2.7508x; 1.0763x over previous
import functools

import jax
import jax.numpy as jnp
from jax import lax
from jax.experimental import pallas as pl
from jax.experimental.pallas import tpu as pltpu
from jax.experimental.pallas import tpu_sc as plsc

D = 64
MAX_NORM = 1.0
L = 16
NC = 2
NS = 16
NW = NC * NS
N_ROWS = 16384
K = 50
NB_TOT = N_ROWS // 128
NB_PER_W = NB_TOT // NW
TILES = K * NB_PER_W
NBUF = 5
LA = 3
GROUPS = 128 // L
SPAD = 17
TPAD = 129


def _rsqrt(s):
    bits = plsc.bitcast(s, jnp.int32)
    r = plsc.bitcast(jnp.int32(0x5F3759DF) - (bits >> 1), jnp.float32)
    half = s * 0.5
    for _ in range(3):
        r = r * (1.5 - half * r * r)
    return r


def _sc_body(idx_hbm, table_hbm, out_hbm, idx_all, bufs, tbufs, s_flat,
             gsems, osems):
    wid = lax.axis_index("s") * NC + lax.axis_index("c")
    nb_base = wid * NB_PER_W
    lanes = lax.iota(jnp.int32, L)
    lanes_pad = lanes * SPAD

    pltpu.sync_copy(
        idx_hbm.at[:, pl.ds(nb_base * 128, NB_PER_W * 128)], idx_all
    )

    def tile_kn(t):
        return t % K, t // K

    def gather_desc(t, b):
        k, nb = tile_kn(t)
        return pltpu.make_async_copy(
            table_hbm.at[idx_all.at[k, pl.ds(nb * 128, 128)]],
            bufs[b],
            gsems[b],
        )

    def out_desc(t, b):
        k, nb = tile_kn(t)
        return pltpu.make_async_copy(
            tbufs[b].at[:, :, pl.ds(0, 128)],
            out_hbm.at[pl.ds(k * 8, 8), nb_base + nb],
            osems[b],
        )

    def compute(b):
        rows = bufs[b]
        tbuf = tbufs[b]
        dvecs = [(c4 * 16 + lanes) for c4 in range(4)]
        dhis = [d >> 3 for d in dvecs]
        dlos = [d & 7 for d in dvecs]

        def grp(g, carry):
            q0 = g * L
            for j in range(L):
                q = q0 + j
                v0 = rows[q, pl.ds(0, 16)]
                v1 = rows[q, pl.ds(16, 16)]
                v2 = rows[q, pl.ds(32, 16)]
                v3 = rows[q, pl.ds(48, 16)]
                s = v0 * v0 + v1 * v1 + v2 * v2 + v3 * v3
                plsc.store_scatter(s_flat, [lanes + (j * SPAD)], s)
            acc = plsc.load_gather(s_flat, [lanes_pad])
            for cc in range(1, L):
                acc = acc + plsc.load_gather(s_flat, [lanes_pad + cc])
            r = _rsqrt(acc)
            norm = acc * r
            scale = jnp.where(
                acc > MAX_NORM * MAX_NORM, MAX_NORM / (norm + 1e-7), 1.0
            )
            for j in range(L):
                q = q0 + j
                sj = scale[j]
                qv = jnp.full((L,), q, jnp.int32)
                for c4 in range(4):
                    v = rows[q, pl.ds(c4 * 16, 16)]
                    plsc.store_scatter(
                        tbuf, [dhis[c4], dlos[c4], qv], v * sj
                    )
            return carry

        lax.fori_loop(0, GROUPS, grp, 0)

    for t0 in range(LA):
        gather_desc(t0, t0).start()

    def body(p, carry):
        for b in range(NBUF):
            t = p * NBUF + b
            gather_desc(t, b).wait()
            compute(b)
            out_desc(t, b).start()
            nb3 = (b + LA) % NBUF

            @pl.when(t >= 2)
            def _():
                out_desc(t - 2, nb3).wait()

            @pl.when(t + LA < TILES)
            def _():
                gather_desc(t + LA, nb3).start()

        return carry

    lax.fori_loop(0, TILES // NBUF, body, 0)
    for t in (TILES - 2, TILES - 1):
        out_desc(t, t % NBUF).wait()


@jax.jit
def kernel(node_idx, table):
    mesh = plsc.VectorSubcoreMesh(core_axis_name="c", subcore_axis_name="s")
    out4 = pl.kernel(
        _sc_body,
        out_type=jax.ShapeDtypeStruct((K * 8, NB_TOT, 8, 128), jnp.float32),
        mesh=mesh,
        compiler_params=pltpu.CompilerParams(
            needs_layout_passes=False, use_tc_tiling_on_sc=False
        ),
        scratch_types=[
            pltpu.VMEM((K, NB_PER_W * 128), jnp.int32),
            [pltpu.VMEM((128, D), jnp.float32) for _ in range(NBUF)],
            [pltpu.VMEM((8, 8, TPAD), jnp.float32) for _ in range(NBUF)],
            pltpu.VMEM((L * SPAD,), jnp.float32),
            [pltpu.SemaphoreType.DMA for _ in range(NBUF)],
            [pltpu.SemaphoreType.DMA for _ in range(NBUF)],
        ],
    )(jnp.transpose(node_idx).astype(jnp.int32), table)
    out5 = out4.reshape(K, 8, NB_TOT, 8, 128)
    return out5.transpose(2, 4, 0, 1, 3).reshape(N_ROWS, K, D)
